# Initial kernel scaffold; baseline (speedup 1.0000x reference)
import jax
import jax.numpy as jnp
from jax.experimental import pallas as pl


def kernel(z_t, num_points, W1, b1, gamma, beta, W2, b2):
    raise NotImplementedError("write your pallas kernel here")


# speedup vs baseline: 3.0358x; 3.0358x over previous
import functools

import jax
import jax.numpy as jnp
from jax import lax
from jax.experimental import pallas as pl
from jax.experimental.pallas import tpu as pltpu
from jax.experimental.pallas import tpu_sc as plsc

_B = 512
_D = 64
_TOTAL = _B * (_B - 1) // 2

_R = 1168
_R2 = _R // 2
_NB = _TOTAL // _R
_W = 64

_NW = 32
_BPW = _TOTAL // _NW
_CH = 128
_NCH = -(-_BPW // _CH)


def _phi_segsum_body(z_ref, w1_ref, b1_ref, g_ref, bt_ref, w2_ref, b2_ref,
                     mdiv_ref, off0_ref, off1_ref, emb_ref, segid_ref,
                     acc_ref):
    i = pl.program_id(0)
    nb = pl.num_programs(0)

    @pl.when(i == 0)
    def _init():
        acc_ref[...] = jnp.zeros_like(acc_ref)

    z = z_ref[...]
    h = jnp.dot(z, w1_ref[...], preferred_element_type=jnp.float32)
    h = h + b1_ref[...]
    mu = jnp.dot(h, mdiv_ref[...], preferred_element_type=jnp.float32)
    c = h - mu
    var = jnp.dot(c * c, mdiv_ref[...], preferred_element_type=jnp.float32)
    h = c * lax.rsqrt(var + 1e-5) * g_ref[...] + bt_ref[...]
    h = jnp.maximum(h, 0.0)
    phi = jnp.dot(h, w2_ref[...], preferred_element_type=jnp.float32)
    phi = phi + b2_ref[...]

    re_ = i * _R + 2 * lax.broadcasted_iota(jnp.int32, (1, _R2), 1)
    ro_ = re_ + 1
    w0 = jnp.sum((off1_ref[...] <= i * _R).astype(jnp.int32))
    w0 = jnp.minimum((w0 // 8) * 8, _B - _W)
    off0w = off0_ref[pl.ds(w0, _W), :]
    off1w = off1_ref[pl.ds(w0, _W), :]
    oh_e = ((re_ >= off0w).astype(jnp.float32)
            - (re_ >= off1w).astype(jnp.float32))
    oh_o = ((ro_ >= off0w).astype(jnp.float32)
            - (ro_ >= off1w).astype(jnp.float32))
    part_e = lax.dot_general(oh_e, phi, (((1,), (0,)), ((), ())),
                             preferred_element_type=jnp.float32)
    part_o = lax.dot_general(oh_o, phi, (((1,), (0,)), ((), ())),
                             preferred_element_type=jnp.float32)
    part_or = jnp.concatenate([part_o[:, _D:], part_o[:, :_D]], axis=1)
    acc_ref[pl.ds(w0, _W), :] += part_e + part_or

    iota_w = lax.broadcasted_iota(jnp.int32, (1, _W), 1).astype(jnp.float32)
    sid_e = w0 + lax.dot_general(iota_w, oh_e, (((1,), (0,)), ((), ())),
                                 preferred_element_type=jnp.float32)
    sid_o = w0 + lax.dot_general(iota_w, oh_o, (((1,), (0,)), ((), ())),
                                 preferred_element_type=jnp.float32)
    segid_ref[...] = jnp.concatenate(
        [sid_e.astype(jnp.int32).reshape(1, 1, _R2),
         sid_o.astype(jnp.int32).reshape(1, 1, _R2)], axis=1)

    @pl.when(i == nb - 1)
    def _finish():
        counts = (off1_ref[...] - off0_ref[...]).astype(jnp.float32)
        emb_ref[...] = acc_ref[:, :_D] / jnp.maximum(counts, 1.0)


_TC_IN_SPECS = [
    pl.BlockSpec((_R2, 2 * _D), lambda i: (i, 0)),
    pl.BlockSpec((2 * _D, 2 * _D), lambda i: (0, 0)),
    pl.BlockSpec((1, 2 * _D), lambda i: (0, 0)),
    pl.BlockSpec((1, 2 * _D), lambda i: (0, 0)),
    pl.BlockSpec((1, 2 * _D), lambda i: (0, 0)),
    pl.BlockSpec((2 * _D, 2 * _D), lambda i: (0, 0)),
    pl.BlockSpec((1, 2 * _D), lambda i: (0, 0)),
    pl.BlockSpec((2 * _D, 2 * _D), lambda i: (0, 0)),
    pl.BlockSpec((_B, 1), lambda i: (0, 0)),
    pl.BlockSpec((_B, 1), lambda i: (0, 0)),
]
_TC_OUT_SPECS = [
    pl.BlockSpec((_B, _D), lambda i: (0, 0)),
    pl.BlockSpec((1, 2, _R2), lambda i: (i, 0, 0)),
]
_TC_OUT_SHAPE = [
    jax.ShapeDtypeStruct((_B, _D), jnp.float32),
    jax.ShapeDtypeStruct((_NB, 2, _R2), jnp.int32),
]

_phi_segsum = pl.pallas_call(
    _phi_segsum_body,
    grid=(_NB,),
    in_specs=_TC_IN_SPECS,
    out_specs=_TC_OUT_SPECS,
    out_shape=_TC_OUT_SHAPE,
    scratch_shapes=[pltpu.VMEM((_B, 2 * _D), jnp.float32)],
    compiler_params=pltpu.CompilerParams(
        dimension_semantics=("arbitrary",),
    ),
)


def _broadcast_sc_body(emb_hbm, segid_hbm, out_hbm, idx_v, buf, sem):
    wid = lax.axis_index("s") * 2 + lax.axis_index("c")
    base = wid * _BPW
    pltpu.sync_copy(segid_hbm.at[pl.ds(base, _BPW)], idx_v)
    for j in range(_NCH):
        cb = j * _CH
        n = min(_CH, _BPW - cb)
        pltpu.async_copy(emb_hbm.at[idx_v.at[pl.ds(cb, n)]],
                         buf.at[pl.ds(0, n)], sem).wait()
        pltpu.sync_copy(buf.at[pl.ds(0, n)],
                        out_hbm.at[pl.ds(base + cb, n)])


@functools.cache
def _broadcast_sc():
    return pl.kernel(
        _broadcast_sc_body,
        mesh=plsc.VectorSubcoreMesh(core_axis_name="c", subcore_axis_name="s"),
        out_type=jax.ShapeDtypeStruct((_TOTAL, _D), jnp.float32),
        scratch_types=[
            pltpu.VMEM((_BPW,), jnp.int32),
            pltpu.VMEM((_CH, _D), jnp.float32),
            pltpu.SemaphoreType.DMA,
        ],
        compiler_params=pltpu.CompilerParams(use_tc_tiling_on_sc=False),
    )


def _tile2(v):
    return jnp.concatenate([v, v]).reshape(1, 2 * _D)


def _blockdiag2(m):
    z = jnp.zeros((_D, _D), m.dtype)
    return jnp.block([[m, z], [z, m]])


def kernel(z_t, num_points, W1, b1, gamma, beta, W2, b2):
    offs = jnp.concatenate(
        [jnp.zeros((1,), jnp.int32), jnp.cumsum(num_points, dtype=jnp.int32)])
    off0 = offs[:_B].reshape(_B, 1)
    off1 = offs[1:].reshape(_B, 1)
    mdiv = _blockdiag2(jnp.full((_D, _D), 1.0 / _D, jnp.float32))
    emb, segid3 = _phi_segsum(
        z_t.reshape(_TOTAL // 2, 2 * _D), _blockdiag2(W1.T), _tile2(b1),
        _tile2(gamma), _tile2(beta), _blockdiag2(W2.T), _tile2(b2),
        mdiv, off0, off1)
    segid = segid3.transpose(0, 2, 1).reshape(_TOTAL)
    return _broadcast_sc()(emb, segid)


# speedup vs baseline: 3.5915x; 1.1830x over previous
import functools

import jax
import jax.numpy as jnp
from jax import lax
from jax.experimental import pallas as pl
from jax.experimental.pallas import tpu as pltpu
from jax.experimental.pallas import tpu_sc as plsc

_B = 512
_D = 64
_TOTAL = _B * (_B - 1) // 2

_R = 1168
_R2 = _R // 2
_NB = _TOTAL // _R
_W = 64

_NW = 32
_NP = _TOTAL // 2
_PCH = 112
_NCHG = _NP // _PCH
_KMAIN = 18


def _phi_segsum_body(z_ref, w1_ref, b1_ref, g_ref, bt_ref, w2_ref, b2_ref,
                     mdiv_ref, off0_ref, off1_ref, emb_ref, segid_ref,
                     acc_ref):
    i = pl.program_id(0)
    nb = pl.num_programs(0)

    @pl.when(i == 0)
    def _init():
        acc_ref[...] = jnp.zeros_like(acc_ref)

    z = z_ref[...]
    h = jnp.dot(z, w1_ref[...], preferred_element_type=jnp.float32)
    h = h + b1_ref[...]
    mu = jnp.dot(h, mdiv_ref[...], preferred_element_type=jnp.float32)
    c = h - mu
    var = jnp.dot(c * c, mdiv_ref[...], preferred_element_type=jnp.float32)
    h = c * lax.rsqrt(var + 1e-5) * g_ref[...] + bt_ref[...]
    h = jnp.maximum(h, 0.0)
    phi = jnp.dot(h, w2_ref[...], preferred_element_type=jnp.float32)
    phi = phi + b2_ref[...]

    re_ = i * _R + 2 * lax.broadcasted_iota(jnp.int32, (1, _R2), 1)
    ro_ = re_ + 1
    w0 = jnp.sum((off1_ref[...] <= i * _R).astype(jnp.int32))
    w0 = jnp.minimum((w0 // 8) * 8, _B - _W)
    off0w = off0_ref[pl.ds(w0, _W), :]
    off1w = off1_ref[pl.ds(w0, _W), :]
    oh_e = ((re_ >= off0w).astype(jnp.float32)
            - (re_ >= off1w).astype(jnp.float32))
    oh_o = ((ro_ >= off0w).astype(jnp.float32)
            - (ro_ >= off1w).astype(jnp.float32))
    part_e = lax.dot_general(oh_e, phi, (((1,), (0,)), ((), ())),
                             preferred_element_type=jnp.float32)
    part_o = lax.dot_general(oh_o, phi, (((1,), (0,)), ((), ())),
                             preferred_element_type=jnp.float32)
    part_or = jnp.concatenate([part_o[:, _D:], part_o[:, :_D]], axis=1)
    acc_ref[pl.ds(w0, _W), :] += part_e + part_or

    iota_w = lax.broadcasted_iota(jnp.int32, (1, _W), 1).astype(jnp.float32)
    sid_sum = 2 * w0 + lax.dot_general(
        iota_w, oh_e + oh_o, (((1,), (0,)), ((), ())),
        preferred_element_type=jnp.float32)
    segid_ref[...] = sid_sum.astype(jnp.int32).reshape(1, 1, _R2)

    @pl.when(i == nb - 1)
    def _finish():
        counts = (off1_ref[...] - off0_ref[...]).astype(jnp.float32)
        v = acc_ref[:, :_D] / jnp.maximum(counts, 1.0)
        vn = jnp.concatenate([v[1:], v[:1]], axis=0)
        a = jnp.concatenate([v, v], axis=1)
        b = jnp.concatenate([v, vn], axis=1)
        emb_ref[...] = jnp.concatenate(
            [a.reshape(_B, 1, 2 * _D), b.reshape(_B, 1, 2 * _D)],
            axis=1).reshape(2 * _B, 2 * _D)


_TC_IN_SPECS = [
    pl.BlockSpec((_R2, 2 * _D), lambda i: (i, 0)),
    pl.BlockSpec((2 * _D, 2 * _D), lambda i: (0, 0)),
    pl.BlockSpec((1, 2 * _D), lambda i: (0, 0)),
    pl.BlockSpec((1, 2 * _D), lambda i: (0, 0)),
    pl.BlockSpec((1, 2 * _D), lambda i: (0, 0)),
    pl.BlockSpec((2 * _D, 2 * _D), lambda i: (0, 0)),
    pl.BlockSpec((1, 2 * _D), lambda i: (0, 0)),
    pl.BlockSpec((2 * _D, 2 * _D), lambda i: (0, 0)),
    pl.BlockSpec((_B, 1), lambda i: (0, 0)),
    pl.BlockSpec((_B, 1), lambda i: (0, 0)),
]
_TC_OUT_SPECS = [
    pl.BlockSpec((2 * _B, 2 * _D), lambda i: (0, 0)),
    pl.BlockSpec((1, 1, _R2), lambda i: (i, 0, 0)),
]
_TC_OUT_SHAPE = [
    jax.ShapeDtypeStruct((2 * _B, 2 * _D), jnp.float32),
    jax.ShapeDtypeStruct((_NB, 1, _R2), jnp.int32),
]

_phi_segsum = pl.pallas_call(
    _phi_segsum_body,
    grid=(_NB,),
    in_specs=_TC_IN_SPECS,
    out_specs=_TC_OUT_SPECS,
    out_shape=_TC_OUT_SHAPE,
    scratch_shapes=[pltpu.VMEM((_B, 2 * _D), jnp.float32)],
    compiler_params=pltpu.CompilerParams(
        dimension_semantics=("arbitrary",),
    ),
)


def _broadcast_sc_body(emb2_hbm, idx_hbm, out2_hbm,
                       idx_v0, idx_v1, buf0, buf1, gsem, ssem):
    w = lax.axis_index("s") * 2 + lax.axis_index("c")
    idxv = (idx_v0, idx_v1)
    bufs = (buf0, buf1)

    def base(k):
        return (w + _NW * k) * _PCH

    g = [None, None]
    s = [None, None]
    for k in range(_KMAIN):
        sl = k % 2
        if k >= 2:
            s[sl].wait()
        pltpu.sync_copy(idx_hbm.at[pl.ds(base(k), _PCH)], idxv[sl])
        g[sl] = pltpu.async_copy(emb2_hbm.at[idxv[sl]], bufs[sl], gsem)
        if k >= 1:
            po = 1 - sl
            g[po].wait()
            s[po] = pltpu.async_copy(
                bufs[po], out2_hbm.at[pl.ds(base(k - 1), _PCH)], ssem)
    last = _KMAIN - 1
    g[last % 2].wait()
    s[last % 2] = pltpu.async_copy(
        bufs[last % 2], out2_hbm.at[pl.ds(base(last), _PCH)], ssem)
    s[0].wait()
    s[1].wait()

    @pl.when(w < _NCHG - _NW * _KMAIN)
    def _tail():
        cb = (_NW * _KMAIN + w) * _PCH
        pltpu.sync_copy(idx_hbm.at[pl.ds(cb, _PCH)], idx_v0)
        pltpu.async_copy(emb2_hbm.at[idx_v0], buf0, gsem).wait()
        pltpu.sync_copy(buf0, out2_hbm.at[pl.ds(cb, _PCH)])


@functools.cache
def _broadcast_sc():
    return pl.kernel(
        _broadcast_sc_body,
        mesh=plsc.VectorSubcoreMesh(core_axis_name="c", subcore_axis_name="s"),
        out_type=jax.ShapeDtypeStruct((_NP, 2 * _D), jnp.float32),
        scratch_types=[
            pltpu.VMEM((_PCH,), jnp.int32),
            pltpu.VMEM((_PCH,), jnp.int32),
            pltpu.VMEM((_PCH, 2 * _D), jnp.float32),
            pltpu.VMEM((_PCH, 2 * _D), jnp.float32),
            pltpu.SemaphoreType.DMA,
            pltpu.SemaphoreType.DMA,
        ],
        compiler_params=pltpu.CompilerParams(use_tc_tiling_on_sc=False),
    )


def _tile2(v):
    return jnp.concatenate([v, v]).reshape(1, 2 * _D)


def _blockdiag2(m):
    z = jnp.zeros((_D, _D), m.dtype)
    return jnp.block([[m, z], [z, m]])


def kernel(z_t, num_points, W1, b1, gamma, beta, W2, b2):
    offs = jnp.concatenate(
        [jnp.zeros((1,), jnp.int32), jnp.cumsum(num_points, dtype=jnp.int32)])
    off0 = offs[:_B].reshape(_B, 1)
    off1 = offs[1:].reshape(_B, 1)
    mdiv = _blockdiag2(jnp.full((_D, _D), 1.0 / _D, jnp.float32))
    emb2, idx3 = _phi_segsum(
        z_t.reshape(_TOTAL // 2, 2 * _D), _blockdiag2(W1.T), _tile2(b1),
        _tile2(gamma), _tile2(beta), _blockdiag2(W2.T), _tile2(b2),
        mdiv, off0, off1)
    idx2 = idx3.reshape(_NP)
    out2 = _broadcast_sc()(emb2, idx2)
    return out2.reshape(_TOTAL, _D)


# speedup vs baseline: 3.8302x; 1.0665x over previous
import functools

import jax
import jax.numpy as jnp
from jax import lax
from jax.experimental import pallas as pl
from jax.experimental.pallas import tpu as pltpu
from jax.experimental.pallas import tpu_sc as plsc

_B = 512
_D = 64
_TOTAL = _B * (_B - 1) // 2

_R = 1168
_R2 = _R // 2
_NB = _TOTAL // _R
_W = 64

_NW = 32
_NP = _TOTAL // 2
_PCH = 112
_NCHG = _NP // _PCH
_KMAIN = 18


def _phi_segsum_body(z_ref, w1_ref, b1_ref, g_ref, bt_ref, w2_ref, b2_ref,
                     mdiv_ref, off0_ref, off1_ref, emb_ref, segid_ref,
                     acc_ref):
    i = pl.program_id(0)
    nb = pl.num_programs(0)

    @pl.when(i == 0)
    def _init():
        acc_ref[...] = jnp.zeros_like(acc_ref)

    z = z_ref[...]
    h = jnp.dot(z, w1_ref[...], preferred_element_type=jnp.float32)
    h = h + b1_ref[...]
    mu = jnp.dot(h, mdiv_ref[...], preferred_element_type=jnp.float32)
    c = h - mu
    var = jnp.dot(c * c, mdiv_ref[...], preferred_element_type=jnp.float32)
    h = c * lax.rsqrt(var + 1e-5) * g_ref[...] + bt_ref[...]
    h = jnp.maximum(h, 0.0)
    phi = jnp.dot(h, w2_ref[...], preferred_element_type=jnp.float32)
    phi = phi + b2_ref[...]

    rows = i * _R + lax.broadcasted_iota(jnp.int32, (1, _R), 1)
    w0 = jnp.sum((off1_ref[...] <= i * _R).astype(jnp.int32))
    w0 = jnp.minimum((w0 // 8) * 8, _B - _W)
    off0w = off0_ref[pl.ds(w0, _W), :]
    off1w = off1_ref[pl.ds(w0, _W), :]
    onehot = ((rows >= off0w).astype(jnp.float32)
              - (rows >= off1w).astype(jnp.float32))
    part = lax.dot_general(onehot, phi, (((1,), (0,)), ((), ())),
                           preferred_element_type=jnp.float32)
    acc_ref[pl.ds(w0, _W), :] += part

    re_ = i * _R + 2 * lax.broadcasted_iota(jnp.int32, (1, _R2), 1)
    ge_sum = ((re_ >= off1w).astype(jnp.float32)
              + (re_ + 1 >= off1w).astype(jnp.float32))
    ones_w = jnp.ones((1, _W), jnp.float32)
    sid_sum = 2 * w0 + lax.dot_general(
        ones_w, ge_sum, (((1,), (0,)), ((), ())),
        preferred_element_type=jnp.float32)
    segid_ref[...] = sid_sum.astype(jnp.int32).reshape(1, 1, _R2)

    @pl.when(i == nb - 1)
    def _finish():
        counts = (off1_ref[...] - off0_ref[...]).astype(jnp.float32)
        v = acc_ref[...] / jnp.maximum(counts, 1.0)
        vn = jnp.concatenate([v[1:], v[:1]], axis=0)
        a = jnp.concatenate([v, v], axis=1)
        b = jnp.concatenate([v, vn], axis=1)
        emb_ref[...] = jnp.concatenate(
            [a.reshape(_B, 1, 2 * _D), b.reshape(_B, 1, 2 * _D)],
            axis=1).reshape(2 * _B, 2 * _D)


_TC_IN_SPECS = [
    pl.BlockSpec((_R, _D), lambda i: (i, 0)),
    pl.BlockSpec((_D, _D), lambda i: (0, 0)),
    pl.BlockSpec((1, _D), lambda i: (0, 0)),
    pl.BlockSpec((1, _D), lambda i: (0, 0)),
    pl.BlockSpec((1, _D), lambda i: (0, 0)),
    pl.BlockSpec((_D, _D), lambda i: (0, 0)),
    pl.BlockSpec((1, _D), lambda i: (0, 0)),
    pl.BlockSpec((_D, _D), lambda i: (0, 0)),
    pl.BlockSpec((_B, 1), lambda i: (0, 0)),
    pl.BlockSpec((_B, 1), lambda i: (0, 0)),
]
_TC_OUT_SPECS = [
    pl.BlockSpec((2 * _B, 2 * _D), lambda i: (0, 0)),
    pl.BlockSpec((1, 1, _R2), lambda i: (i, 0, 0)),
]
_TC_OUT_SHAPE = [
    jax.ShapeDtypeStruct((2 * _B, 2 * _D), jnp.float32),
    jax.ShapeDtypeStruct((_NB, 1, _R2), jnp.int32),
]

_phi_segsum = pl.pallas_call(
    _phi_segsum_body,
    grid=(_NB,),
    in_specs=_TC_IN_SPECS,
    out_specs=_TC_OUT_SPECS,
    out_shape=_TC_OUT_SHAPE,
    scratch_shapes=[pltpu.VMEM((_B, _D), jnp.float32)],
    compiler_params=pltpu.CompilerParams(
        dimension_semantics=("arbitrary",),
    ),
)


def _broadcast_sc_body(emb2_hbm, idx_hbm, out2_hbm,
                       idx_v0, idx_v1, buf0, buf1, gsem, ssem):
    w = lax.axis_index("s") * 2 + lax.axis_index("c")
    idxv = (idx_v0, idx_v1)
    bufs = (buf0, buf1)

    def base(k):
        return (w + _NW * k) * _PCH

    g = [None, None]
    s = [None, None]
    for k in range(_KMAIN):
        sl = k % 2
        if k >= 2:
            s[sl].wait()
        pltpu.sync_copy(idx_hbm.at[pl.ds(base(k), _PCH)], idxv[sl])
        g[sl] = pltpu.async_copy(emb2_hbm.at[idxv[sl]], bufs[sl], gsem)
        if k >= 1:
            po = 1 - sl
            g[po].wait()
            s[po] = pltpu.async_copy(
                bufs[po], out2_hbm.at[pl.ds(base(k - 1), _PCH)], ssem)
    last = _KMAIN - 1
    g[last % 2].wait()
    s[last % 2] = pltpu.async_copy(
        bufs[last % 2], out2_hbm.at[pl.ds(base(last), _PCH)], ssem)
    s[0].wait()
    s[1].wait()

    @pl.when(w < _NCHG - _NW * _KMAIN)
    def _tail():
        cb = (_NW * _KMAIN + w) * _PCH
        pltpu.sync_copy(idx_hbm.at[pl.ds(cb, _PCH)], idx_v0)
        pltpu.async_copy(emb2_hbm.at[idx_v0], buf0, gsem).wait()
        pltpu.sync_copy(buf0, out2_hbm.at[pl.ds(cb, _PCH)])


@functools.cache
def _broadcast_sc():
    return pl.kernel(
        _broadcast_sc_body,
        mesh=plsc.VectorSubcoreMesh(core_axis_name="c", subcore_axis_name="s"),
        out_type=jax.ShapeDtypeStruct((_NP, 2 * _D), jnp.float32),
        scratch_types=[
            pltpu.VMEM((_PCH,), jnp.int32),
            pltpu.VMEM((_PCH,), jnp.int32),
            pltpu.VMEM((_PCH, 2 * _D), jnp.float32),
            pltpu.VMEM((_PCH, 2 * _D), jnp.float32),
            pltpu.SemaphoreType.DMA,
            pltpu.SemaphoreType.DMA,
        ],
        compiler_params=pltpu.CompilerParams(use_tc_tiling_on_sc=False),
    )


def kernel(z_t, num_points, W1, b1, gamma, beta, W2, b2):
    offs = jnp.concatenate(
        [jnp.zeros((1,), jnp.int32), jnp.cumsum(num_points, dtype=jnp.int32)])
    off0 = offs[:_B].reshape(_B, 1)
    off1 = offs[1:].reshape(_B, 1)
    mdiv = jnp.full((_D, _D), 1.0 / _D, jnp.float32)
    emb2, idx3 = _phi_segsum(
        z_t, W1.T, b1.reshape(1, _D),
        gamma.reshape(1, _D), beta.reshape(1, _D), W2.T, b2.reshape(1, _D),
        mdiv, off0, off1)
    idx2 = idx3.reshape(_NP)
    out2 = _broadcast_sc()(emb2, idx2)
    return out2.reshape(_TOTAL, _D)


# speedup vs baseline: 4.6984x; 1.2267x over previous
import functools

import jax
import jax.numpy as jnp
from jax import lax
from jax.experimental import pallas as pl
from jax.experimental.pallas import tpu as pltpu
from jax.experimental.pallas import tpu_sc as plsc

_B = 512
_D = 64
_TOTAL = _B * (_B - 1) // 2

_R = 1792
_NB = _TOTAL // _R
_W = 64

_NW = 32
_BPW = _TOTAL // _NW
_CH = 128
_SIZES = [_CH] * 31 + [_BPW - 31 * _CH]
_NCH = len(_SIZES)
_RING = 4


def _phi_segsum_body(z_ref, w1_ref, b1_ref, g_ref, bt_ref, w2_ref, b2_ref,
                     mdiv_ref, off0_ref, off1_ref, emb_ref, segid_ref,
                     acc_ref):
    i = pl.program_id(0)
    nb = pl.num_programs(0)

    @pl.when(i == 0)
    def _init():
        acc_ref[...] = jnp.zeros_like(acc_ref)

    z = z_ref[...]
    h = jnp.dot(w1_ref[...], z, preferred_element_type=jnp.float32)
    h = h + b1_ref[...]
    mu = jnp.dot(mdiv_ref[...], h, preferred_element_type=jnp.float32)
    c = h - mu
    var = jnp.dot(mdiv_ref[...], c * c, preferred_element_type=jnp.float32)
    h = c * lax.rsqrt(var + 1e-5) * g_ref[...] + bt_ref[...]
    h = jnp.maximum(h, 0.0)
    phi = jnp.dot(w2_ref[...], h, preferred_element_type=jnp.float32)
    phi = phi + b2_ref[...]

    rows = i * _R + lax.broadcasted_iota(jnp.int32, (1, _R), 1)
    w0 = jnp.sum((off1_ref[...] <= i * _R).astype(jnp.int32))
    w0 = jnp.minimum((w0 // 8) * 8, _B - _W)
    off0w = off0_ref[pl.ds(w0, _W), :]
    off1w = off1_ref[pl.ds(w0, _W), :]
    onehot = ((rows >= off0w).astype(jnp.float32)
              - (rows >= off1w).astype(jnp.float32))
    part = lax.dot_general(onehot, phi, (((1,), (1,)), ((), ())),
                           preferred_element_type=jnp.float32)
    acc_ref[pl.ds(w0, _W), :] += part

    iota_w = lax.broadcasted_iota(jnp.int32, (1, _W), 1).astype(jnp.float32)
    sid = w0 + lax.dot_general(iota_w, onehot, (((1,), (0,)), ((), ())),
                               preferred_element_type=jnp.float32)
    segid_ref[...] = sid.astype(jnp.int32).reshape(1, 1, _R)

    @pl.when(i == nb - 1)
    def _finish():
        counts = (off1_ref[...] - off0_ref[...]).astype(jnp.float32)
        emb_ref[...] = acc_ref[...] / jnp.maximum(counts, 1.0)


_TC_IN_SPECS = [
    pl.BlockSpec((_D, _R), lambda i: (0, i)),
    pl.BlockSpec((_D, _D), lambda i: (0, 0)),
    pl.BlockSpec((_D, 1), lambda i: (0, 0)),
    pl.BlockSpec((_D, 1), lambda i: (0, 0)),
    pl.BlockSpec((_D, 1), lambda i: (0, 0)),
    pl.BlockSpec((_D, _D), lambda i: (0, 0)),
    pl.BlockSpec((_D, 1), lambda i: (0, 0)),
    pl.BlockSpec((_D, _D), lambda i: (0, 0)),
    pl.BlockSpec((_B, 1), lambda i: (0, 0)),
    pl.BlockSpec((_B, 1), lambda i: (0, 0)),
]
_TC_OUT_SPECS = [
    pl.BlockSpec((_B, _D), lambda i: (0, 0)),
    pl.BlockSpec((1, 1, _R), lambda i: (i, 0, 0)),
]
_TC_OUT_SHAPE = [
    jax.ShapeDtypeStruct((_B, _D), jnp.float32),
    jax.ShapeDtypeStruct((_NB, 1, _R), jnp.int32),
]

_phi_segsum = pl.pallas_call(
    _phi_segsum_body,
    grid=(_NB,),
    in_specs=_TC_IN_SPECS,
    out_specs=_TC_OUT_SPECS,
    out_shape=_TC_OUT_SHAPE,
    scratch_shapes=[pltpu.VMEM((_B, _D), jnp.float32)],
    compiler_params=pltpu.CompilerParams(
        dimension_semantics=("arbitrary",),
    ),
)


def _broadcast_sc_body(emb_hbm, idx_hbm, out_hbm,
                       idx_v, b0, b1, b2, b3, gsem, ssem):
    w = lax.axis_index("s") * 2 + lax.axis_index("c")
    base = w * _BPW
    bufs = (b0, b1, b2, b3)
    pltpu.sync_copy(idx_hbm.at[pl.ds(base, _BPW)], idx_v)
    g = [None] * _RING
    s = [None] * _RING
    for k in range(_NCH):
        sl = k % _RING
        if k >= _RING:
            s[sl].wait()
        n = _SIZES[k]
        g[sl] = pltpu.async_copy(
            emb_hbm.at[idx_v.at[pl.ds(k * _CH, n)]],
            bufs[sl].at[pl.ds(0, n)], gsem)
        if k >= 1:
            pk = (k - 1) % _RING
            pn = _SIZES[k - 1]
            g[pk].wait()
            s[pk] = pltpu.async_copy(
                bufs[pk].at[pl.ds(0, pn)],
                out_hbm.at[pl.ds(base + (k - 1) * _CH, pn)], ssem)
    last = _NCH - 1
    sl = last % _RING
    g[sl].wait()
    s[sl] = pltpu.async_copy(
        bufs[sl].at[pl.ds(0, _SIZES[last])],
        out_hbm.at[pl.ds(base + last * _CH, _SIZES[last])], ssem)
    for t in range(_RING):
        s[t].wait()


@functools.cache
def _broadcast_sc():
    return pl.kernel(
        _broadcast_sc_body,
        mesh=plsc.VectorSubcoreMesh(core_axis_name="c", subcore_axis_name="s"),
        out_type=jax.ShapeDtypeStruct((_TOTAL, _D), jnp.float32),
        scratch_types=[
            pltpu.VMEM((_BPW,), jnp.int32),
            pltpu.VMEM((_CH, _D), jnp.float32),
            pltpu.VMEM((_CH, _D), jnp.float32),
            pltpu.VMEM((_CH, _D), jnp.float32),
            pltpu.VMEM((_CH, _D), jnp.float32),
            pltpu.SemaphoreType.DMA,
            pltpu.SemaphoreType.DMA,
        ],
        compiler_params=pltpu.CompilerParams(use_tc_tiling_on_sc=False),
    )


def kernel(z_t, num_points, W1, b1, gamma, beta, W2, b2):
    offs = jnp.concatenate(
        [jnp.zeros((1,), jnp.int32), jnp.cumsum(num_points, dtype=jnp.int32)])
    off0 = offs[:_B].reshape(_B, 1)
    off1 = offs[1:].reshape(_B, 1)
    mdiv = jnp.full((_D, _D), 1.0 / _D, jnp.float32)
    emb, segid3 = _phi_segsum(
        z_t.T, W1, b1.reshape(_D, 1),
        gamma.reshape(_D, 1), beta.reshape(_D, 1), W2, b2.reshape(_D, 1),
        mdiv, off0, off1)
    segid = segid3.reshape(_TOTAL)
    return _broadcast_sc()(emb, segid)


# speedup vs baseline: 7.4069x; 1.5765x over previous
import functools

import jax
import jax.numpy as jnp
from jax import lax
from jax.experimental import pallas as pl
from jax.experimental.pallas import tpu as pltpu
from jax.experimental.pallas import tpu_sc as plsc

_B = 512
_D = 64
_TOTAL = _B * (_B - 1) // 2

_R = 1792
_NB = _TOTAL // _R
_W = 64

_NW = 32
_BPW = _TOTAL // _NW
_CH = 128
_SIZES = [_CH] * 31 + [_BPW - 31 * _CH]
_NCH = len(_SIZES)
_RING = 4


def _phi_segsum_body(z_ref, w1_ref, b1_ref, g_ref, bt_ref, w2_ref, b2_ref,
                     mdiv_ref, off0_ref, off1_ref, emb_ref, segid_ref,
                     acc_ref):
    i = pl.program_id(0)
    nb = pl.num_programs(0)

    @pl.when(i == 0)
    def _init():
        acc_ref[...] = jnp.zeros_like(acc_ref)

    z = z_ref[...]
    h = jnp.dot(w1_ref[...], z, preferred_element_type=jnp.float32)
    h = h + b1_ref[...]
    mu = jnp.dot(mdiv_ref[...], h, preferred_element_type=jnp.float32)
    c = h - mu
    var = jnp.dot(mdiv_ref[...], c * c, preferred_element_type=jnp.float32)
    h = c * lax.rsqrt(var + 1e-5) * g_ref[...] + bt_ref[...]
    h = jnp.maximum(h, 0.0)
    phi = jnp.dot(w2_ref[...], h, preferred_element_type=jnp.float32)
    phi = phi + b2_ref[...]

    rows = i * _R + lax.broadcasted_iota(jnp.int32, (1, _R), 1)
    w0 = jnp.sum((off1_ref[...] <= i * _R).astype(jnp.int32))
    w0 = jnp.minimum((w0 // 8) * 8, _B - _W)
    off0w = off0_ref[pl.ds(w0, _W), :]
    off1w = off1_ref[pl.ds(w0, _W), :]
    onehot = ((rows >= off0w).astype(jnp.float32)
              - (rows >= off1w).astype(jnp.float32))
    part = lax.dot_general(onehot, phi, (((1,), (1,)), ((), ())),
                           preferred_element_type=jnp.float32)
    acc_ref[pl.ds(w0, _W), :] += part

    iota_w = lax.broadcasted_iota(jnp.int32, (1, _W), 1).astype(jnp.float32)
    sid = w0 + lax.dot_general(iota_w, onehot, (((1,), (0,)), ((), ())),
                               preferred_element_type=jnp.float32)
    segid_ref[...] = sid.astype(jnp.int32).reshape(1, 1, _R)

    @pl.when(i == nb - 1)
    def _finish():
        counts = (off1_ref[...] - off0_ref[...]).astype(jnp.float32)
        emb_ref[...] = acc_ref[...] / jnp.maximum(counts, 1.0)


_TC_IN_SPECS = [
    pl.BlockSpec((_D, _R), lambda i: (0, i)),
    pl.BlockSpec((_D, _D), lambda i: (0, 0)),
    pl.BlockSpec((_D, 1), lambda i: (0, 0)),
    pl.BlockSpec((_D, 1), lambda i: (0, 0)),
    pl.BlockSpec((_D, 1), lambda i: (0, 0)),
    pl.BlockSpec((_D, _D), lambda i: (0, 0)),
    pl.BlockSpec((_D, 1), lambda i: (0, 0)),
    pl.BlockSpec((_D, _D), lambda i: (0, 0)),
    pl.BlockSpec((_B, 1), lambda i: (0, 0)),
    pl.BlockSpec((_B, 1), lambda i: (0, 0)),
]
_TC_OUT_SPECS = [
    pl.BlockSpec((_B, _D), lambda i: (0, 0)),
    pl.BlockSpec((1, 1, _R), lambda i: (i, 0, 0)),
]
_TC_OUT_SHAPE = [
    jax.ShapeDtypeStruct((_B, _D), jnp.float32),
    jax.ShapeDtypeStruct((_NB, 1, _R), jnp.int32),
]

_phi_segsum = pl.pallas_call(
    _phi_segsum_body,
    grid=(_NB,),
    in_specs=_TC_IN_SPECS,
    out_specs=_TC_OUT_SPECS,
    out_shape=_TC_OUT_SHAPE,
    scratch_shapes=[pltpu.VMEM((_B, _D), jnp.float32)],
    compiler_params=pltpu.CompilerParams(
        dimension_semantics=("arbitrary",),
    ),
)


def _broadcast_sc_body(emb_hbm, idx_hbm, out_hbm,
                       idx_v, emb_sp, b0, b1, b2, b3, gsem, ssem):
    w = lax.axis_index("s") * 2 + lax.axis_index("c")
    base = w * _BPW
    bufs = (b0, b1, b2, b3)

    @pl.when(lax.axis_index("s") == 0)
    def _stage():
        pltpu.sync_copy(emb_hbm, emb_sp)

    pltpu.sync_copy(idx_hbm.at[pl.ds(base, _BPW)], idx_v)
    plsc.subcore_barrier()
    g = [None] * _RING
    s = [None] * _RING
    for k in range(_NCH):
        sl = k % _RING
        if k >= _RING:
            s[sl].wait()
        n = _SIZES[k]
        g[sl] = pltpu.async_copy(
            emb_sp.at[idx_v.at[pl.ds(k * _CH, n)]],
            bufs[sl].at[pl.ds(0, n)], gsem)
        if k >= 1:
            pk = (k - 1) % _RING
            pn = _SIZES[k - 1]
            g[pk].wait()
            s[pk] = pltpu.async_copy(
                bufs[pk].at[pl.ds(0, pn)],
                out_hbm.at[pl.ds(base + (k - 1) * _CH, pn)], ssem)
    last = _NCH - 1
    sl = last % _RING
    g[sl].wait()
    s[sl] = pltpu.async_copy(
        bufs[sl].at[pl.ds(0, _SIZES[last])],
        out_hbm.at[pl.ds(base + last * _CH, _SIZES[last])], ssem)
    for t in range(_RING):
        s[t].wait()


@functools.cache
def _broadcast_sc():
    return pl.kernel(
        _broadcast_sc_body,
        mesh=plsc.VectorSubcoreMesh(core_axis_name="c", subcore_axis_name="s"),
        out_type=jax.ShapeDtypeStruct((_TOTAL, _D), jnp.float32),
        scratch_types=[
            pltpu.VMEM((_BPW,), jnp.int32),
            pltpu.VMEM_SHARED((_B, _D), jnp.float32),
            pltpu.VMEM((_CH, _D), jnp.float32),
            pltpu.VMEM((_CH, _D), jnp.float32),
            pltpu.VMEM((_CH, _D), jnp.float32),
            pltpu.VMEM((_CH, _D), jnp.float32),
            pltpu.SemaphoreType.DMA,
            pltpu.SemaphoreType.DMA,
        ],
        compiler_params=pltpu.CompilerParams(use_tc_tiling_on_sc=False),
    )


def kernel(z_t, num_points, W1, b1, gamma, beta, W2, b2):
    offs = jnp.concatenate(
        [jnp.zeros((1,), jnp.int32), jnp.cumsum(num_points, dtype=jnp.int32)])
    off0 = offs[:_B].reshape(_B, 1)
    off1 = offs[1:].reshape(_B, 1)
    mdiv = jnp.full((_D, _D), 1.0 / _D, jnp.float32)
    emb, segid3 = _phi_segsum(
        z_t.T, W1, b1.reshape(_D, 1),
        gamma.reshape(_D, 1), beta.reshape(_D, 1), W2, b2.reshape(_D, 1),
        mdiv, off0, off1)
    segid = segid3.reshape(_TOTAL)
    return _broadcast_sc()(emb, segid)
